# baseline (speedup 1.0000x reference)
import functools

import jax
import jax.numpy as jnp
from jax import lax
from jax.experimental import pallas as pl
from jax.experimental.pallas import tpu as pltpu
from jax.experimental.pallas import tpu_sc as plsc

NC = 2
NS = 16
NW = NC * NS
CH = 1024
SW = 128


_P = lax.Precision.HIGHEST


def _sc_mesh():
    return plsc.VectorSubcoreMesh(
        core_axis_name="c", subcore_axis_name="s", num_cores=NC, num_subcores=NS
    )


def _tile_copy_split(sid, n, copy_fn):
    rpt8 = ((n // NS) // 8) * 8 + 8
    last = n - 15 * rpt8

    @pl.when(sid < NS - 1)
    def _():
        copy_fn(pl.multiple_of(sid * rpt8, 8), rpt8)

    @pl.when(sid == NS - 1)
    def _():
        copy_fn(15 * rpt8, last)


def _sc_deg(dst2d, ones_tbl, zeros_tbl, n, npad, k_chunks):
    tot = k_chunks * CH // SW

    @functools.partial(
        pl.kernel,
        out_type=jax.ShapeDtypeStruct((NC, n, 2), jnp.float32),
        mesh=_sc_mesh(),
        scratch_types=[
            pltpu.VMEM((tot, SW), jnp.int32),
            pltpu.VMEM((SW, 2), jnp.float32),
            pltpu.VMEM_SHARED((npad, 2), jnp.float32),
            pltpu.SemaphoreType.DMA,
        ],
        compiler_params=pltpu.CompilerParams(use_tc_tiling_on_sc=False),
    )
    def k(dst_hbm, ones_hbm, zeros_hbm, out_hbm, idx_d, ones_v, acc, sem):
        cid = lax.axis_index("c")
        sid = lax.axis_index("s")

        def seed(start, size):
            @pl.when(cid == 0)
            def _():
                pltpu.sync_copy(ones_hbm.at[pl.ds(start, size)],
                                acc.at[pl.ds(start, size)])

            @pl.when(cid == 1)
            def _():
                pltpu.sync_copy(zeros_hbm.at[pl.ds(start, size)],
                                acc.at[pl.ds(start, size)])

        _tile_copy_split(sid, n, seed)
        pltpu.sync_copy(ones_hbm.at[pl.ds(0, SW)], ones_v)
        wid = sid * NC + cid
        pltpu.sync_copy(dst_hbm.at[pl.ds(wid * tot, tot)], idx_d)
        plsc.subcore_barrier()

        for j in range(tot):
            pltpu.async_copy(ones_v, acc.at[idx_d.at[j]], sem, add=True)
        for j in range(tot):
            pltpu.make_async_copy(ones_v, acc.at[idx_d.at[j]], sem).wait()

        plsc.subcore_barrier()
        _tile_copy_split(sid, n, lambda start, size: pltpu.sync_copy(
            acc.at[pl.ds(start, size)], out_hbm.at[cid, pl.ds(start, size)]))

    return k(dst2d, ones_tbl, zeros_tbl)


def _sc_edge(src2d, dst2d, y, zeros_tbl, n, npad, k_chunks, h):
    tot = k_chunks * CH // SW
    NB = 3
    LA = 2

    @functools.partial(
        pl.kernel,
        out_type=jax.ShapeDtypeStruct((NC, n, h), jnp.float32),
        mesh=_sc_mesh(),
        scratch_types=[
            pltpu.VMEM((tot, SW), jnp.int32),
            pltpu.VMEM((tot, SW), jnp.int32),
            pltpu.VMEM((NB * SW, h), jnp.float32),
            pltpu.VMEM_SHARED((npad, h), jnp.float32),
            pltpu.VMEM_SHARED((n, h), jnp.float32),
            pltpu.SemaphoreType.DMA((NB,)),
            pltpu.SemaphoreType.DMA((NB,)),
        ],
        compiler_params=pltpu.CompilerParams(use_tc_tiling_on_sc=False),
    )
    def k(src_hbm, dst_hbm, y_hbm, zeros_hbm, out_hbm,
          idx_s, idx_d, rows, acc, y_spm, gsem, ssem):
        cid = lax.axis_index("c")
        sid = lax.axis_index("s")
        wid = sid * NC + cid
        r0 = wid * tot

        pltpu.sync_copy(src_hbm.at[pl.ds(r0, tot)], idx_s)
        pltpu.sync_copy(dst_hbm.at[pl.ds(r0, tot)], idx_d)
        _tile_copy_split(sid, n, lambda start, size: pltpu.sync_copy(
            y_hbm.at[pl.ds(start, size)], y_spm.at[pl.ds(start, size)]))
        _tile_copy_split(sid, n, lambda start, size: pltpu.sync_copy(
            zeros_hbm.at[pl.ds(start, size)], acc.at[pl.ds(start, size)]))
        plsc.subcore_barrier()

        def gather(j, issue):
            m = j % NB
            cp = pltpu.make_async_copy(
                y_spm.at[idx_s.at[j]], rows.at[pl.ds(m * SW, SW)], gsem.at[m])
            if issue:
                cp.start()
            else:
                cp.wait()

        def scatter(j, issue):
            m = j % NB
            if issue:
                pltpu.async_copy(rows.at[pl.ds(m * SW, SW)],
                                 acc.at[idx_d.at[j]], ssem.at[m], add=True)
            else:
                pltpu.make_async_copy(rows.at[pl.ds(m * SW, SW)],
                                      acc.at[idx_d.at[j]], ssem.at[m]).wait()

        for j in range(tot + LA):
            if j < tot:
                if j >= NB:
                    scatter(j - NB, False)
                gather(j, True)
            jj = j - LA
            if 0 <= jj < tot:
                gather(jj, False)
                scatter(jj, True)
        for j in range(tot - NB, tot):
            scatter(j, False)

        plsc.subcore_barrier()
        _tile_copy_split(sid, n, lambda start, size: pltpu.sync_copy(
            acc.at[pl.ds(start, size)], out_hbm.at[cid, pl.ds(start, size)]))

    return k(src2d, dst2d, y, zeros_tbl)


def _tc_prep(x, w1, d0, d1, n, bn):
    d = x.shape[1]
    hh = w1.shape[1]
    grid = n // bn

    def body(x_ref, w_ref, d0_ref, d1_ref, y_ref, dinv_ref):
        deg = d0_ref[:, :1] + d1_ref[:, :1]
        dinv = lax.rsqrt(deg)
        xw = jnp.dot(x_ref[...], w_ref[...],
                     preferred_element_type=jnp.float32)
        y_ref[...] = xw * dinv
        dinv_ref[...] = dinv

    return pl.pallas_call(
        body,
        grid=(grid,),
        in_specs=[
            pl.BlockSpec((bn, d), lambda i: (i, 0)),
            pl.BlockSpec((d, hh), lambda i: (0, 0)),
            pl.BlockSpec((bn, 2), lambda i: (i, 0)),
            pl.BlockSpec((bn, 2), lambda i: (i, 0)),
        ],
        out_specs=[
            pl.BlockSpec((bn, hh), lambda i: (i, 0)),
            pl.BlockSpec((bn, 1), lambda i: (i, 0)),
        ],
        out_shape=[
            jax.ShapeDtypeStruct((n, hh), jnp.float32),
            jax.ShapeDtypeStruct((n, 1), jnp.float32),
        ],
    )(x, w1, d0, d1)


def _tc_mid(p0, p1, y, dinv, b, w, n, bn):
    hh = w.shape[1]
    grid = n // bn

    def body(p0_ref, p1_ref, y_ref, dinv_ref, b_ref, w_ref, o_ref):
        dinv = dinv_ref[...]
        hcur = jnp.maximum(
            (p0_ref[...] + p1_ref[...] + y_ref[...]) * dinv + b_ref[0:1, :], 0.0)
        o_ref[...] = jnp.dot(hcur, w_ref[...],
                             preferred_element_type=jnp.float32) * dinv

    return pl.pallas_call(
        body,
        grid=(grid,),
        in_specs=[
            pl.BlockSpec((bn, hh), lambda i: (i, 0)),
            pl.BlockSpec((bn, hh), lambda i: (i, 0)),
            pl.BlockSpec((bn, hh), lambda i: (i, 0)),
            pl.BlockSpec((bn, 1), lambda i: (i, 0)),
            pl.BlockSpec((8, hh), lambda i: (0, 0)),
            pl.BlockSpec((hh, hh), lambda i: (0, 0)),
        ],
        out_specs=pl.BlockSpec((bn, hh), lambda i: (i, 0)),
        out_shape=jax.ShapeDtypeStruct((n, hh), jnp.float32),
    )(p0, p1, y, dinv, b, w)


def _tc_final(p0, p1, y, dinv, b, batch2d, wfc, bfc, n, bn, g, c):
    hh = wfc.shape[0]
    grid = n // bn

    def body(p0_ref, p1_ref, y_ref, dinv_ref, b_ref, batch_ref, wfc_ref, bfc_ref,
             o_ref, seg_acc, cnt_acc):
        i = pl.program_id(0)

        @pl.when(i == 0)
        def _():
            seg_acc[...] = jnp.zeros_like(seg_acc)
            cnt_acc[...] = jnp.zeros_like(cnt_acc)

        hcur = (p0_ref[...] + p1_ref[...] + y_ref[...]) * dinv_ref[...] + b_ref[0:1, :]
        gids = batch_ref[...]
        oh = (gids == lax.broadcasted_iota(jnp.int32, (1, g), 1)).astype(jnp.float32)
        seg_acc[...] += lax.dot_general(oh, hcur, (((0,), (0,)), ((), ())),
                                        precision=_P,
                                        preferred_element_type=jnp.float32)
        cnt_acc[...] += lax.dot_general(oh, jnp.ones((bn, 1), jnp.float32),
                                        (((0,), (0,)), ((), ())),
                                        precision=_P,
                                        preferred_element_type=jnp.float32)

        @pl.when(i == grid - 1)
        def _():
            pooled = seg_acc[...] / jnp.maximum(cnt_acc[...], 1.0)
            o_ref[...] = jnp.dot(pooled, wfc_ref[...], precision=_P,
                                 preferred_element_type=jnp.float32) + bfc_ref[0:1, :]

    return pl.pallas_call(
        body,
        grid=(grid,),
        in_specs=[
            pl.BlockSpec((bn, hh), lambda i: (i, 0)),
            pl.BlockSpec((bn, hh), lambda i: (i, 0)),
            pl.BlockSpec((bn, hh), lambda i: (i, 0)),
            pl.BlockSpec((bn, 1), lambda i: (i, 0)),
            pl.BlockSpec((8, hh), lambda i: (0, 0)),
            pl.BlockSpec((bn, 1), lambda i: (i, 0)),
            pl.BlockSpec((hh, c), lambda i: (0, 0)),
            pl.BlockSpec((8, c), lambda i: (0, 0)),
        ],
        out_specs=pl.BlockSpec((g, c), lambda i: (0, 0)),
        out_shape=jax.ShapeDtypeStruct((g, c), jnp.float32),
        scratch_shapes=[
            pltpu.VMEM((g, hh), jnp.float32),
            pltpu.VMEM((g, 1), jnp.float32),
        ],
    )(p0, p1, y, dinv, b, batch2d, wfc, bfc)


def kernel(x, edge_index, batch, W1, b1, W2, b2, W3, b3, Wfc, bfc):
    n, d = x.shape
    e = edge_index.shape[1]
    hh = W1.shape[1]
    g = 64
    c = Wfc.shape[1]
    bn = n

    per_tile = -(-e // NW)
    k_chunks = -(-per_tile // CH)
    e_pad = NW * k_chunks * CH
    pad = e_pad - e
    src = jnp.concatenate([edge_index[0], jnp.zeros((pad,), jnp.int32)])
    dst = jnp.concatenate([edge_index[1], jnp.full((pad,), n, jnp.int32)])
    src2d = src.reshape(e_pad // SW, SW)
    dst2d = dst.reshape(e_pad // SW, SW)

    npad = n + 240
    ones_tbl = jnp.ones((n, 2), jnp.float32)
    zeros8 = jnp.zeros((n, 2), jnp.float32)
    zeros_tbl = jnp.zeros((n, hh), jnp.float32)

    degp = _sc_deg(dst2d, ones_tbl, zeros8, n, npad, k_chunks)
    y1, dinv = _tc_prep(x, W1, degp[0], degp[1], n, bn)

    b1r = jnp.broadcast_to(b1[None, :], (8, hh))
    b2r = jnp.broadcast_to(b2[None, :], (8, hh))
    b3r = jnp.broadcast_to(b3[None, :], (8, hh))
    bfcr = jnp.broadcast_to(bfc[None, :], (8, c))

    p = _sc_edge(src2d, dst2d, y1, zeros_tbl, n, npad, k_chunks, hh)
    y2 = _tc_mid(p[0], p[1], y1, dinv, b1r, W2, n, bn)
    p = _sc_edge(src2d, dst2d, y2, zeros_tbl, n, npad, k_chunks, hh)
    y3 = _tc_mid(p[0], p[1], y2, dinv, b2r, W3, n, bn)
    p = _sc_edge(src2d, dst2d, y3, zeros_tbl, n, npad, k_chunks, hh)

    batch2d = batch.reshape(n, 1)
    return _tc_final(p[0], p[1], y3, dinv, b3r, batch2d, Wfc, bfcr, n, bn, g, c)


# speedup vs baseline: 1.0186x; 1.0186x over previous
import functools

import jax
import jax.numpy as jnp
from jax import lax
from jax.experimental import pallas as pl
from jax.experimental.pallas import tpu as pltpu
from jax.experimental.pallas import tpu_sc as plsc

NC = 2
NS = 16
NW = NC * NS
CH = 1024
SW = 128


_P = lax.Precision.HIGHEST


def _sc_mesh():
    return plsc.VectorSubcoreMesh(
        core_axis_name="c", subcore_axis_name="s", num_cores=NC, num_subcores=NS
    )


def _tile_copy_split(sid, n, copy_fn):
    rpt8 = ((n // NS) // 8) * 8 + 8
    last = n - 15 * rpt8

    @pl.when(sid < NS - 1)
    def _():
        copy_fn(pl.multiple_of(sid * rpt8, 8), rpt8)

    @pl.when(sid == NS - 1)
    def _():
        copy_fn(15 * rpt8, last)


def _sc_deg(dst2d, ones_tbl, zeros_tbl, n, npad, k_chunks):
    tot = k_chunks * CH // SW

    @functools.partial(
        pl.kernel,
        out_type=jax.ShapeDtypeStruct((NC, n, 8), jnp.float32),
        mesh=_sc_mesh(),
        scratch_types=[
            pltpu.VMEM((tot, SW), jnp.int32),
            pltpu.VMEM((SW, 8), jnp.float32),
            pltpu.VMEM_SHARED((npad, 8), jnp.float32),
            pltpu.SemaphoreType.DMA,
        ],
        compiler_params=pltpu.CompilerParams(use_tc_tiling_on_sc=False),
    )
    def k(dst_hbm, ones_hbm, zeros_hbm, out_hbm, idx_d, ones_v, acc, sem):
        cid = lax.axis_index("c")
        sid = lax.axis_index("s")

        def seed(start, size):
            @pl.when(cid == 0)
            def _():
                pltpu.sync_copy(ones_hbm.at[pl.ds(start, size)],
                                acc.at[pl.ds(start, size)])

            @pl.when(cid == 1)
            def _():
                pltpu.sync_copy(zeros_hbm.at[pl.ds(start, size)],
                                acc.at[pl.ds(start, size)])

        _tile_copy_split(sid, n, seed)
        pltpu.sync_copy(ones_hbm.at[pl.ds(0, SW)], ones_v)
        wid = sid * NC + cid
        pltpu.sync_copy(dst_hbm.at[pl.ds(wid * tot, tot)], idx_d)
        plsc.subcore_barrier()

        for j in range(tot):
            pltpu.async_copy(ones_v, acc.at[idx_d.at[j]], sem, add=True)
        for j in range(tot):
            pltpu.make_async_copy(ones_v, acc.at[idx_d.at[j]], sem).wait()

        plsc.subcore_barrier()
        _tile_copy_split(sid, n, lambda start, size: pltpu.sync_copy(
            acc.at[pl.ds(start, size)], out_hbm.at[cid, pl.ds(start, size)]))

    return k(dst2d, ones_tbl, zeros_tbl)


def _sc_edge(src2d, dst2d, y, zeros_tbl, n, npad, k_chunks, h):
    tot = k_chunks * CH // SW
    NB = 3
    LA = 2

    @functools.partial(
        pl.kernel,
        out_type=jax.ShapeDtypeStruct((NC, n, h), jnp.float32),
        mesh=_sc_mesh(),
        scratch_types=[
            pltpu.VMEM((tot, SW), jnp.int32),
            pltpu.VMEM((tot, SW), jnp.int32),
            pltpu.VMEM((NB * SW, h), jnp.float32),
            pltpu.VMEM_SHARED((npad, h), jnp.float32),
            pltpu.VMEM_SHARED((n, h), jnp.float32),
            pltpu.SemaphoreType.DMA((NB,)),
            pltpu.SemaphoreType.DMA((NB,)),
        ],
        compiler_params=pltpu.CompilerParams(use_tc_tiling_on_sc=False),
    )
    def k(src_hbm, dst_hbm, y_hbm, zeros_hbm, out_hbm,
          idx_s, idx_d, rows, acc, y_spm, gsem, ssem):
        cid = lax.axis_index("c")
        sid = lax.axis_index("s")
        wid = sid * NC + cid
        r0 = wid * tot

        pltpu.sync_copy(src_hbm.at[pl.ds(r0, tot)], idx_s)
        pltpu.sync_copy(dst_hbm.at[pl.ds(r0, tot)], idx_d)
        _tile_copy_split(sid, n, lambda start, size: pltpu.sync_copy(
            y_hbm.at[pl.ds(start, size)], y_spm.at[pl.ds(start, size)]))
        _tile_copy_split(sid, n, lambda start, size: pltpu.sync_copy(
            zeros_hbm.at[pl.ds(start, size)], acc.at[pl.ds(start, size)]))
        plsc.subcore_barrier()

        def gather(j, issue):
            m = j % NB
            cp = pltpu.make_async_copy(
                y_spm.at[idx_s.at[j]], rows.at[pl.ds(m * SW, SW)], gsem.at[m])
            if issue:
                cp.start()
            else:
                cp.wait()

        def scatter(j, issue):
            m = j % NB
            if issue:
                pltpu.async_copy(rows.at[pl.ds(m * SW, SW)],
                                 acc.at[idx_d.at[j]], ssem.at[m], add=True)
            else:
                pltpu.make_async_copy(rows.at[pl.ds(m * SW, SW)],
                                      acc.at[idx_d.at[j]], ssem.at[m]).wait()

        for j in range(tot + LA):
            if j < tot:
                if j >= NB:
                    scatter(j - NB, False)
                gather(j, True)
            jj = j - LA
            if 0 <= jj < tot:
                gather(jj, False)
                scatter(jj, True)
        for j in range(tot - NB, tot):
            scatter(j, False)

        plsc.subcore_barrier()
        _tile_copy_split(sid, n, lambda start, size: pltpu.sync_copy(
            acc.at[pl.ds(start, size)], out_hbm.at[cid, pl.ds(start, size)]))

    return k(src2d, dst2d, y, zeros_tbl)


def _tc_prep(x, w1, d0, d1, n, bn):
    d = x.shape[1]
    hh = w1.shape[1]
    grid = n // bn

    def body(x_ref, w_ref, d0_ref, d1_ref, y_ref, dinv_ref):
        deg = d0_ref[:, :1] + d1_ref[:, :1]
        dinv = lax.rsqrt(deg)
        xw = jnp.dot(x_ref[...], w_ref[...],
                     preferred_element_type=jnp.float32)
        y_ref[...] = xw * dinv
        dinv_ref[...] = dinv

    return pl.pallas_call(
        body,
        grid=(grid,),
        in_specs=[
            pl.BlockSpec((bn, d), lambda i: (i, 0)),
            pl.BlockSpec((d, hh), lambda i: (0, 0)),
            pl.BlockSpec((bn, 8), lambda i: (i, 0)),
            pl.BlockSpec((bn, 8), lambda i: (i, 0)),
        ],
        out_specs=[
            pl.BlockSpec((bn, hh), lambda i: (i, 0)),
            pl.BlockSpec((bn, 1), lambda i: (i, 0)),
        ],
        out_shape=[
            jax.ShapeDtypeStruct((n, hh), jnp.float32),
            jax.ShapeDtypeStruct((n, 1), jnp.float32),
        ],
    )(x, w1, d0, d1)


def _tc_mid(p0, p1, y, dinv, b, w, n, bn):
    hh = w.shape[1]
    grid = n // bn

    def body(p0_ref, p1_ref, y_ref, dinv_ref, b_ref, w_ref, o_ref):
        dinv = dinv_ref[...]
        hcur = jnp.maximum(
            (p0_ref[...] + p1_ref[...] + y_ref[...]) * dinv + b_ref[0:1, :], 0.0)
        o_ref[...] = jnp.dot(hcur, w_ref[...],
                             preferred_element_type=jnp.float32) * dinv

    return pl.pallas_call(
        body,
        grid=(grid,),
        in_specs=[
            pl.BlockSpec((bn, hh), lambda i: (i, 0)),
            pl.BlockSpec((bn, hh), lambda i: (i, 0)),
            pl.BlockSpec((bn, hh), lambda i: (i, 0)),
            pl.BlockSpec((bn, 1), lambda i: (i, 0)),
            pl.BlockSpec((8, hh), lambda i: (0, 0)),
            pl.BlockSpec((hh, hh), lambda i: (0, 0)),
        ],
        out_specs=pl.BlockSpec((bn, hh), lambda i: (i, 0)),
        out_shape=jax.ShapeDtypeStruct((n, hh), jnp.float32),
    )(p0, p1, y, dinv, b, w)


def _tc_final(p0, p1, y, dinv, b, batch2d, wfc, bfc, n, bn, g, c):
    hh = wfc.shape[0]
    grid = n // bn

    def body(p0_ref, p1_ref, y_ref, dinv_ref, b_ref, batch_ref, wfc_ref, bfc_ref,
             o_ref, seg_acc, cnt_acc):
        i = pl.program_id(0)

        @pl.when(i == 0)
        def _():
            seg_acc[...] = jnp.zeros_like(seg_acc)
            cnt_acc[...] = jnp.zeros_like(cnt_acc)

        hcur = (p0_ref[...] + p1_ref[...] + y_ref[...]) * dinv_ref[...] + b_ref[0:1, :]
        gids = batch_ref[...]
        oh = (gids == lax.broadcasted_iota(jnp.int32, (1, g), 1)).astype(jnp.float32)
        seg_acc[...] += lax.dot_general(oh, hcur, (((0,), (0,)), ((), ())),
                                        precision=_P,
                                        preferred_element_type=jnp.float32)
        cnt_acc[...] += lax.dot_general(oh, jnp.ones((bn, 1), jnp.float32),
                                        (((0,), (0,)), ((), ())),
                                        precision=_P,
                                        preferred_element_type=jnp.float32)

        @pl.when(i == grid - 1)
        def _():
            pooled = seg_acc[...] / jnp.maximum(cnt_acc[...], 1.0)
            o_ref[...] = jnp.dot(pooled, wfc_ref[...], precision=_P,
                                 preferred_element_type=jnp.float32) + bfc_ref[0:1, :]

    return pl.pallas_call(
        body,
        grid=(grid,),
        in_specs=[
            pl.BlockSpec((bn, hh), lambda i: (i, 0)),
            pl.BlockSpec((bn, hh), lambda i: (i, 0)),
            pl.BlockSpec((bn, hh), lambda i: (i, 0)),
            pl.BlockSpec((bn, 1), lambda i: (i, 0)),
            pl.BlockSpec((8, hh), lambda i: (0, 0)),
            pl.BlockSpec((bn, 1), lambda i: (i, 0)),
            pl.BlockSpec((hh, c), lambda i: (0, 0)),
            pl.BlockSpec((8, c), lambda i: (0, 0)),
        ],
        out_specs=pl.BlockSpec((g, c), lambda i: (0, 0)),
        out_shape=jax.ShapeDtypeStruct((g, c), jnp.float32),
        scratch_shapes=[
            pltpu.VMEM((g, hh), jnp.float32),
            pltpu.VMEM((g, 1), jnp.float32),
        ],
    )(p0, p1, y, dinv, b, batch2d, wfc, bfc)


def kernel(x, edge_index, batch, W1, b1, W2, b2, W3, b3, Wfc, bfc):
    n, d = x.shape
    e = edge_index.shape[1]
    hh = W1.shape[1]
    g = 64
    c = Wfc.shape[1]
    bn = n

    per_tile = -(-e // NW)
    k_chunks = -(-per_tile // CH)
    e_pad = NW * k_chunks * CH
    pad = e_pad - e
    src = jnp.concatenate([edge_index[0], jnp.zeros((pad,), jnp.int32)])
    dst = jnp.concatenate([edge_index[1], jnp.full((pad,), n, jnp.int32)])
    src2d = src.reshape(e_pad // SW, SW)
    dst2d = dst.reshape(e_pad // SW, SW)

    npad = n + 240
    ones_tbl = jnp.ones((n, 8), jnp.float32)
    zeros8 = jnp.zeros((n, 8), jnp.float32)
    zeros_tbl = jnp.zeros((n, hh), jnp.float32)

    degp = _sc_deg(dst2d, ones_tbl, zeros8, n, npad, k_chunks)
    y1, dinv = _tc_prep(x, W1, degp[0], degp[1], n, bn)

    b1r = jnp.broadcast_to(b1[None, :], (8, hh))
    b2r = jnp.broadcast_to(b2[None, :], (8, hh))
    b3r = jnp.broadcast_to(b3[None, :], (8, hh))
    bfcr = jnp.broadcast_to(bfc[None, :], (8, c))

    p = _sc_edge(src2d, dst2d, y1, zeros_tbl, n, npad, k_chunks, hh)
    y2 = _tc_mid(p[0], p[1], y1, dinv, b1r, W2, n, bn)
    p = _sc_edge(src2d, dst2d, y2, zeros_tbl, n, npad, k_chunks, hh)
    y3 = _tc_mid(p[0], p[1], y2, dinv, b2r, W3, n, bn)
    p = _sc_edge(src2d, dst2d, y3, zeros_tbl, n, npad, k_chunks, hh)

    batch2d = batch.reshape(n, 1)
    return _tc_final(p[0], p[1], y3, dinv, b3r, batch2d, Wfc, bfcr, n, bn, g, c)
